# baseline (speedup 1.0000x reference)
import functools

import jax
import jax.numpy as jnp
import numpy as np
from jax.experimental import pallas as pl
from jax.experimental.pallas import tpu as pltpu

TASKS = 3
EPS = float(np.finfo(np.float64).eps)


def _moe_kernel(x_ref, wg_ref, w1_ref, b1_ref, w2_ref, b2_ref,
                out_ref, gates_ref, yacc_ref, acc_ref, sem,
                *, n_experts, n_hc, bt):
    i = pl.program_id(0)
    e = pl.program_id(1)
    hc = pl.program_id(2)

    @pl.when(jnp.logical_and(e == 0, hc == 0))
    def _compute_gates():
        x = x_ref[...]
        for t in range(TASKS):
            logits = jax.lax.dot_general(
                x, wg_ref[t],
                (((1,), (0,)), ((), ())),
                preferred_element_type=jnp.float32)
            idx = jax.lax.broadcasted_iota(jnp.int32, logits.shape, 1)
            m1 = jnp.max(logits, axis=-1, keepdims=True)
            eq1 = logits == m1
            i1 = jnp.min(jnp.where(eq1, idx, 127), axis=-1, keepdims=True)
            first1 = idx == i1
            l2 = jnp.where(first1, -jnp.inf, logits)
            m2 = jnp.max(l2, axis=-1, keepdims=True)
            eq2 = l2 == m2
            i2 = jnp.min(jnp.where(eq2, idx, 127), axis=-1, keepdims=True)
            first2 = idx == i2
            z = jnp.exp(m2 - m1)
            g1 = 1.0 / (1.0 + z)
            g2 = z / (1.0 + z)
            gates = jnp.where(first1, g1, 0.0) + jnp.where(first2, g2, 0.0)
            gates = jnp.where(gates <= 0.0001, 0.0, gates)
            gates_ref[t] = gates

    x = x_ref[...]
    w1 = w1_ref[0]
    w2 = w2_ref[0]
    h = jax.lax.dot_general(x, w1, (((1,), (1,)), ((), ())),
                            preferred_element_type=jnp.float32)
    h = jax.nn.relu(h + b1_ref[0])
    part = jax.lax.dot_general(h, w2, (((1,), (1,)), ((), ())),
                               preferred_element_type=jnp.float32)

    @pl.when(hc == 0)
    def _y_init():
        yacc_ref[...] = part

    @pl.when(hc > 0)
    def _y_acc():
        yacc_ref[...] += part

    @pl.when(hc == n_hc - 1)
    def _combine():
        y = yacc_ref[...] + b2_ref[0]
        ey = jnp.exp(y)
        eidx = jax.lax.broadcasted_iota(jnp.int32, gates_ref.shape, 2)
        for t in range(TASKS):
            ge = jnp.sum(jnp.where(eidx[t] == e, gates_ref[t], 0.0),
                         axis=-1, keepdims=True)
            contrib = ge * ey

            @pl.when(e == 0)
            def _init():
                acc_ref[t] = contrib

            @pl.when(e > 0)
            def _acc():
                acc_ref[t] += contrib

        @pl.when(e == n_experts - 1)
        def _finish():
            for t in range(TASKS):
                a = acc_ref[t]
                acc_ref[t] = jnp.log(jnp.where(a == 0.0, EPS, a))
            cp = pltpu.make_async_copy(
                acc_ref, out_ref.at[:, pl.ds(i * bt, bt), :], sem)
            cp.start()
            cp.wait()


def kernel(x, w_gate, fc1_w, fc1_b, fc2_w, fc2_b):
    B, D = x.shape
    E, H, _ = fc1_w.shape
    O = fc2_w.shape[1]
    BT = 1024
    HC = 1024
    n_b = B // BT
    n_hc = H // HC

    grid = (n_b, E, n_hc)
    out = pl.pallas_call(
        functools.partial(_moe_kernel, n_experts=E, n_hc=n_hc, bt=BT),
        grid=grid,
        in_specs=[
            pl.BlockSpec((BT, D), lambda i, e, hc: (i, 0)),
            pl.BlockSpec((TASKS, D, E), lambda i, e, hc: (0, 0, 0)),
            pl.BlockSpec((1, HC, D), lambda i, e, hc: (e, hc, 0)),
            pl.BlockSpec((1, 1, HC), lambda i, e, hc: (e, 0, hc)),
            pl.BlockSpec((1, O, HC), lambda i, e, hc: (e, 0, hc)),
            pl.BlockSpec((1, 1, O), lambda i, e, hc: (e, 0, 0)),
        ],
        out_specs=pl.BlockSpec(memory_space=pltpu.MemorySpace.HBM),
        out_shape=jax.ShapeDtypeStruct((TASKS, B, O), jnp.float32),
        scratch_shapes=[
            pltpu.VMEM((TASKS, BT, E), jnp.float32),
            pltpu.VMEM((BT, O), jnp.float32),
            pltpu.VMEM((TASKS, BT, O), jnp.float32),
            pltpu.SemaphoreType.DMA,
        ],
        compiler_params=pltpu.CompilerParams(
            vmem_limit_bytes=63 * 1024 * 1024),
    )(x, w_gate, fc1_w, fc1_b.reshape(E, 1, H), fc2_w, fc2_b.reshape(E, 1, O))
    return out


# speedup vs baseline: 1.0067x; 1.0067x over previous
import functools

import jax
import jax.numpy as jnp
import numpy as np
from jax.experimental import pallas as pl
from jax.experimental.pallas import tpu as pltpu

TASKS = 3
EPS = float(np.finfo(np.float64).eps)


def _moe_kernel(x_ref, wg_ref, w1_ref, b1_ref, w2_ref, b2_ref,
                out_ref, gates_ref, yacc_ref, acc_ref, sem,
                *, n_experts, n_hc, bt):
    i = pl.program_id(0)
    e = pl.program_id(1)
    hc = pl.program_id(2)

    @pl.when(jnp.logical_and(e == 0, hc == 0))
    def _compute_gates():
        x = x_ref[...]
        for t in range(TASKS):
            logits = jax.lax.dot_general(
                x, wg_ref[t],
                (((1,), (0,)), ((), ())),
                preferred_element_type=jnp.float32)
            idx = jax.lax.broadcasted_iota(jnp.int32, logits.shape, 1)
            m1 = jnp.max(logits, axis=-1, keepdims=True)
            eq1 = logits == m1
            i1 = jnp.min(jnp.where(eq1, idx, 127), axis=-1, keepdims=True)
            first1 = idx == i1
            l2 = jnp.where(first1, -jnp.inf, logits)
            m2 = jnp.max(l2, axis=-1, keepdims=True)
            eq2 = l2 == m2
            i2 = jnp.min(jnp.where(eq2, idx, 127), axis=-1, keepdims=True)
            first2 = idx == i2
            z = jnp.exp(m2 - m1)
            g1 = 1.0 / (1.0 + z)
            g2 = z / (1.0 + z)
            gates = jnp.where(first1, g1, 0.0) + jnp.where(first2, g2, 0.0)
            gates = jnp.where(gates <= 0.0001, 0.0, gates)
            gates_ref[t] = gates

    x = x_ref[...].astype(jnp.bfloat16)
    w1 = w1_ref[0].astype(jnp.bfloat16)
    w2 = w2_ref[0].astype(jnp.bfloat16)
    h = jax.lax.dot_general(x, w1, (((1,), (1,)), ((), ())),
                            preferred_element_type=jnp.float32)
    h = jax.nn.relu(h + b1_ref[0]).astype(jnp.bfloat16)
    part = jax.lax.dot_general(h, w2, (((1,), (1,)), ((), ())),
                               preferred_element_type=jnp.float32)

    @pl.when(hc == 0)
    def _y_init():
        yacc_ref[...] = part

    @pl.when(hc > 0)
    def _y_acc():
        yacc_ref[...] += part

    @pl.when(hc == n_hc - 1)
    def _combine():
        y = yacc_ref[...] + b2_ref[0]
        ey = jnp.exp(y)
        eidx = jax.lax.broadcasted_iota(jnp.int32, gates_ref.shape, 2)
        for t in range(TASKS):
            ge = jnp.sum(jnp.where(eidx[t] == e, gates_ref[t], 0.0),
                         axis=-1, keepdims=True)
            contrib = ge * ey

            @pl.when(e == 0)
            def _init():
                acc_ref[t] = contrib

            @pl.when(e > 0)
            def _acc():
                acc_ref[t] += contrib

        @pl.when(e == n_experts - 1)
        def _finish():
            for t in range(TASKS):
                a = acc_ref[t]
                acc_ref[t] = jnp.log(jnp.where(a == 0.0, EPS, a))
            cp = pltpu.make_async_copy(
                acc_ref, out_ref.at[:, pl.ds(i * bt, bt), :], sem)
            cp.start()
            cp.wait()


def kernel(x, w_gate, fc1_w, fc1_b, fc2_w, fc2_b):
    B, D = x.shape
    E, H, _ = fc1_w.shape
    O = fc2_w.shape[1]
    BT = 1024
    HC = 1024
    n_b = B // BT
    n_hc = H // HC

    grid = (n_b, E, n_hc)
    out = pl.pallas_call(
        functools.partial(_moe_kernel, n_experts=E, n_hc=n_hc, bt=BT),
        grid=grid,
        in_specs=[
            pl.BlockSpec((BT, D), lambda i, e, hc: (i, 0)),
            pl.BlockSpec((TASKS, D, E), lambda i, e, hc: (0, 0, 0)),
            pl.BlockSpec((1, HC, D), lambda i, e, hc: (e, hc, 0)),
            pl.BlockSpec((1, 1, HC), lambda i, e, hc: (e, 0, hc)),
            pl.BlockSpec((1, O, HC), lambda i, e, hc: (e, 0, hc)),
            pl.BlockSpec((1, 1, O), lambda i, e, hc: (e, 0, 0)),
        ],
        out_specs=pl.BlockSpec(memory_space=pltpu.MemorySpace.HBM),
        out_shape=jax.ShapeDtypeStruct((TASKS, B, O), jnp.float32),
        scratch_shapes=[
            pltpu.VMEM((TASKS, BT, E), jnp.float32),
            pltpu.VMEM((BT, O), jnp.float32),
            pltpu.VMEM((TASKS, BT, O), jnp.float32),
            pltpu.SemaphoreType.DMA,
        ],
        compiler_params=pltpu.CompilerParams(
            vmem_limit_bytes=63 * 1024 * 1024),
    )(x, w_gate, fc1_w, fc1_b.reshape(E, 1, H), fc2_w, fc2_b.reshape(E, 1, O))
    return out
